# baseline (speedup 1.0000x reference)
import functools
import math

import jax
import jax.numpy as jnp
from jax import lax
from jax.experimental import pallas as pl
from jax.experimental.pallas import tpu as pltpu
from jax.experimental.pallas import tpu_sc as plsc

SCALE = 4.0
THRESHOLD = 0.2
_LOGIT_T = math.log(THRESHOLD / (1.0 - THRESHOLD))

N_INST, H, W = 64, 512, 512
NC, NS, L = 2, 16, 16
NW = NC * NS
IPW = N_INST // NW
CHUNK_ROWS = 64
CHUNKS_PER_INST = H // CHUNK_ROWS
CHUNK_WORDS = CHUNK_ROWS * W
TOTAL_CHUNKS = IPW * CHUNKS_PER_INST


def _masses_kernel(logits_hbm, out_hbm, buf0, buf1, out_v, sem0, sem1):
    wid = lax.axis_index("s") * NC + lax.axis_index("c")
    worker_base = wid * (IPW * H * W)

    bufs = (buf0, buf1)
    sems = (sem0, sem1)

    def start(g):
        return pltpu.async_copy(
            logits_hbm.at[pl.ds(worker_base + g * CHUNK_WORDS, CHUNK_WORDS)],
            bufs[g % 2],
            sems[g % 2],
        )

    lane_x0 = lax.iota(jnp.int32, L).astype(jnp.float32)
    zero_v = jnp.zeros((L,), jnp.float32)
    one_v = jnp.ones((L,), jnp.float32)

    def chunk_body(buf, row0, cnt, sx, sy):
        def row_body(r, carry):
            cnt, sx, sy = carry
            off = r * W
            rc = zero_v
            rx = zero_v
            for j in range(W // L):
                v = buf[pl.ds(off + j * L, L)]
                m = v > _LOGIT_T
                rc = rc + jnp.where(m, one_v, zero_v)
                rx = rx + jnp.where(m, lane_x0 + float(j * L), zero_v)
            y = (row0 + r).astype(jnp.float32)
            return (cnt + rc, sx + rx, sy + y * rc)

        return lax.fori_loop(0, CHUNK_ROWS, row_body, (cnt, sx, sy))

    cp = start(0)
    acc = (zero_v, zero_v, zero_v)
    for g in range(TOTAL_CHUNKS):
        nxt = start(g + 1) if g + 1 < TOTAL_CHUNKS else None
        cp.wait()
        row0 = (g % CHUNKS_PER_INST) * CHUNK_ROWS
        acc = chunk_body(bufs[g % 2], row0, *acc)
        if (g + 1) % CHUNKS_PER_INST == 0:
            inst_i = g // CHUNKS_PER_INST
            for k, vec in enumerate(acc):
                out_v[pl.ds((inst_i * 3 + k) * L, L)] = vec
            acc = (zero_v, zero_v, zero_v)
        cp = nxt

    pltpu.sync_copy(out_v, out_hbm.at[pl.ds(wid * (IPW * 3 * L), IPW * 3 * L)])


_masses = functools.partial(
    pl.kernel,
    out_type=jax.ShapeDtypeStruct((N_INST * 3 * L,), jnp.float32),
    mesh=plsc.VectorSubcoreMesh(core_axis_name="c", subcore_axis_name="s"),
    scratch_types=[
        pltpu.VMEM((CHUNK_WORDS,), jnp.float32),
        pltpu.VMEM((CHUNK_WORDS,), jnp.float32),
        pltpu.VMEM((IPW * 3 * L,), jnp.float32),
        pltpu.SemaphoreType.DMA,
        pltpu.SemaphoreType.DMA,
    ],
)(_masses_kernel)


def kernel(logits, mean_depths, K):
    n = logits.shape[0]
    lanes = _masses(logits.reshape(-1)).reshape(n, 3, L).sum(axis=-1)
    counts, sum_x, sum_y = lanes[:, 0], lanes[:, 1], lanes[:, 2]
    denom = jnp.maximum(counts, 1.0)
    mean_x = sum_x / denom * SCALE
    mean_y = sum_y / denom * SCALE
    ones = jnp.ones((n,), jnp.float32)
    xy1 = jnp.stack([mean_x, mean_y, ones], axis=1)
    Kinv = jnp.linalg.inv(K)
    return (xy1 @ Kinv.T) * mean_depths


# speedup vs baseline: 1.1006x; 1.1006x over previous
import functools
import math

import jax
import jax.numpy as jnp
from jax import lax
from jax.experimental import pallas as pl
from jax.experimental.pallas import tpu as pltpu
from jax.experimental.pallas import tpu_sc as plsc

SCALE = 4.0
THRESHOLD = 0.2
_LOGIT_T = math.log(THRESHOLD / (1.0 - THRESHOLD))

N_INST, H, W = 64, 512, 512
NC, NS, L = 2, 16, 16
NW = NC * NS
IPW = N_INST // NW
CHUNK_ROWS = 64
CHUNKS_PER_INST = H // CHUNK_ROWS
CHUNK_WORDS = CHUNK_ROWS * W
TOTAL_CHUNKS = IPW * CHUNKS_PER_INST


def _masses_kernel(logits_hbm, out_hbm, buf0, buf1, out_v, sem0, sem1):
    wid = lax.axis_index("s") * NC + lax.axis_index("c")
    worker_base = wid * (IPW * H * W)

    bufs = (buf0, buf1)
    sems = (sem0, sem1)

    def start(g):
        return pltpu.async_copy(
            logits_hbm.at[pl.ds(worker_base + g * CHUNK_WORDS, CHUNK_WORDS)],
            bufs[g % 2],
            sems[g % 2],
        )

    lane_x0 = lax.iota(jnp.int32, L).astype(jnp.float32)
    zero_v = jnp.zeros((L,), jnp.float32)
    PACK = float(1 << 18)
    INV_PACK = 1.0 / PACK
    NACC = 4
    GROUPS = W // L // NACC

    def chunk_body(buf, row0, cnt, sx, sy):
        def row_body(r, carry):
            cnt, sx, sy = carry
            off = r * W
            s = [None] * NACC
            xv = [lane_x0 + (PACK + 16.0 * k) for k in range(NACC)]
            for g in range(GROUPS):
                for k in range(NACC):
                    v = buf[pl.ds(off + (g * NACC + k) * L, L)]
                    m = v > _LOGIT_T
                    t = jnp.where(m, xv[k], zero_v)
                    s[k] = t if g == 0 else s[k] + t
                if g + 1 < GROUPS:
                    xv = [x + float(NACC * L) for x in xv]
            stot = (s[0] + s[1]) + (s[2] + s[3])
            rcf = (stot * INV_PACK).astype(jnp.int32).astype(jnp.float32)
            rx = stot - rcf * PACK
            y = (row0 + r).astype(jnp.float32)
            return (cnt + rcf, sx + rx, sy + y * rcf)

        return lax.fori_loop(0, CHUNK_ROWS, row_body, (cnt, sx, sy))

    cp = start(0)
    acc = (zero_v, zero_v, zero_v)
    for g in range(TOTAL_CHUNKS):
        nxt = start(g + 1) if g + 1 < TOTAL_CHUNKS else None
        cp.wait()
        row0 = (g % CHUNKS_PER_INST) * CHUNK_ROWS
        acc = chunk_body(bufs[g % 2], row0, *acc)
        if (g + 1) % CHUNKS_PER_INST == 0:
            inst_i = g // CHUNKS_PER_INST
            for k, vec in enumerate(acc):
                out_v[pl.ds((inst_i * 3 + k) * L, L)] = vec
            acc = (zero_v, zero_v, zero_v)
        cp = nxt

    pltpu.sync_copy(out_v, out_hbm.at[pl.ds(wid * (IPW * 3 * L), IPW * 3 * L)])


_masses = functools.partial(
    pl.kernel,
    out_type=jax.ShapeDtypeStruct((N_INST * 3 * L,), jnp.float32),
    mesh=plsc.VectorSubcoreMesh(core_axis_name="c", subcore_axis_name="s"),
    scratch_types=[
        pltpu.VMEM((CHUNK_WORDS,), jnp.float32),
        pltpu.VMEM((CHUNK_WORDS,), jnp.float32),
        pltpu.VMEM((IPW * 3 * L,), jnp.float32),
        pltpu.SemaphoreType.DMA,
        pltpu.SemaphoreType.DMA,
    ],
)(_masses_kernel)


def kernel(logits, mean_depths, K):
    n = logits.shape[0]
    lanes = _masses(logits.reshape(-1)).reshape(n, 3, L).sum(axis=-1)
    counts, sum_x, sum_y = lanes[:, 0], lanes[:, 1], lanes[:, 2]
    denom = jnp.maximum(counts, 1.0)
    mean_x = sum_x / denom * SCALE
    mean_y = sum_y / denom * SCALE
    ones = jnp.ones((n,), jnp.float32)
    xy1 = jnp.stack([mean_x, mean_y, ones], axis=1)
    Kinv = jnp.linalg.inv(K)
    return (xy1 @ Kinv.T) * mean_depths


# speedup vs baseline: 2.0788x; 1.8888x over previous
import functools
import math

import jax
import jax.numpy as jnp
from jax import lax
from jax.experimental import pallas as pl
from jax.experimental.pallas import tpu as pltpu
from jax.experimental.pallas import tpu_sc as plsc

SCALE = 4.0
THRESHOLD = 0.2
_LOGIT_T = math.log(THRESHOLD / (1.0 - THRESHOLD))

N_INST, H, W = 64, 512, 512
NC, NS, L = 2, 16, 16
NW = NC * NS
IPW = N_INST // NW
CHUNK_ROWS = 64
CHUNKS_PER_INST = H // CHUNK_ROWS
CHUNK_WORDS = CHUNK_ROWS * W
TOTAL_CHUNKS = IPW * CHUNKS_PER_INST


def _masses_kernel(logits_hbm, out_hbm, buf0, buf1, out_v, sem0, sem1):
    wid = lax.axis_index("s") * NC + lax.axis_index("c")
    worker_row0 = wid * (IPW * H)

    bufs = (buf0, buf1)
    sems = (sem0, sem1)

    def start(g):
        return pltpu.async_copy(
            logits_hbm.at[pl.ds(worker_row0 + g * CHUNK_ROWS, CHUNK_ROWS)],
            bufs[g % 2],
            sems[g % 2],
        )

    lane_x0 = lax.iota(jnp.int32, L).astype(jnp.float32)
    zero_v = jnp.zeros((L,), jnp.float32)
    PACK = float(1 << 18)
    INV_PACK = 1.0 / PACK
    NACC = 4
    GROUPS = W // L // NACC

    def chunk_body(buf, row0, cnt, sx, sy):
        def row_body(r, carry):
            cnt, sx, sy = carry
            s = [None] * NACC
            xv = [lane_x0 + (PACK + 16.0 * k) for k in range(NACC)]
            for g in range(GROUPS):
                for k in range(NACC):
                    v = buf[r, pl.ds((g * NACC + k) * L, L)]
                    m = v > _LOGIT_T
                    t = jnp.where(m, xv[k], zero_v)
                    s[k] = t if g == 0 else s[k] + t
                if g + 1 < GROUPS:
                    xv = [x + float(NACC * L) for x in xv]
            stot = (s[0] + s[1]) + (s[2] + s[3])
            rcf = (stot * INV_PACK).astype(jnp.int32).astype(jnp.float32)
            rx = stot - rcf * PACK
            y = (row0 + r).astype(jnp.float32)
            return (cnt + rcf, sx + rx, sy + y * rcf)

        return lax.fori_loop(0, CHUNK_ROWS, row_body, (cnt, sx, sy))

    cp = start(0)
    acc = (zero_v, zero_v, zero_v)
    for g in range(TOTAL_CHUNKS):
        nxt = start(g + 1) if g + 1 < TOTAL_CHUNKS else None
        cp.wait()
        row0 = (g % CHUNKS_PER_INST) * CHUNK_ROWS
        acc = chunk_body(bufs[g % 2], row0, *acc)
        if (g + 1) % CHUNKS_PER_INST == 0:
            inst_i = g // CHUNKS_PER_INST
            for k, vec in enumerate(acc):
                out_v[pl.ds((inst_i * 3 + k) * L, L)] = vec
            acc = (zero_v, zero_v, zero_v)
        cp = nxt

    pltpu.sync_copy(out_v, out_hbm.at[pl.ds(wid * (IPW * 3 * L), IPW * 3 * L)])


_masses = functools.partial(
    pl.kernel,
    out_type=jax.ShapeDtypeStruct((N_INST * 3 * L,), jnp.float32),
    mesh=plsc.VectorSubcoreMesh(core_axis_name="c", subcore_axis_name="s"),
    scratch_types=[
        pltpu.VMEM((CHUNK_ROWS, W), jnp.float32),
        pltpu.VMEM((CHUNK_ROWS, W), jnp.float32),
        pltpu.VMEM((IPW * 3 * L,), jnp.float32),
        pltpu.SemaphoreType.DMA,
        pltpu.SemaphoreType.DMA,
    ],
    compiler_params=pltpu.CompilerParams(use_tc_tiling_on_sc=True),
)(_masses_kernel)


def kernel(logits, mean_depths, K):
    n = logits.shape[0]
    lanes = _masses(logits.reshape(n * H, W)).reshape(n, 3, L).sum(axis=-1)
    counts, sum_x, sum_y = lanes[:, 0], lanes[:, 1], lanes[:, 2]
    denom = jnp.maximum(counts, 1.0)
    mean_x = sum_x / denom * SCALE
    mean_y = sum_y / denom * SCALE
    ones = jnp.ones((n,), jnp.float32)
    xy1 = jnp.stack([mean_x, mean_y, ones], axis=1)
    Kinv = jnp.linalg.inv(K)
    return (xy1 @ Kinv.T) * mean_depths
